# baseline (speedup 1.0000x reference)
import functools

import jax
import jax.numpy as jnp
from jax.experimental import pallas as pl
from jax.experimental.pallas import tpu as pltpu

VOCAB = 32000
DIM = 2048
N_LAYERS = 2
N_DENSE = 1
N_HEADS = 16
QK_NOPE = 128
QK_ROPE = 32
V_HEAD = 128
KV_RANK = 512
INTER = 4096
MOE_INTER = 512
N_EXP = 32
TOPK = 4
N_SHARED = 2
ROPE_THETA = 10000.0
EPS = 1e-6
CAP = 512
S = 2048

_F32 = jnp.float32


_BF16 = jnp.bfloat16


def _bdot(a, b):
    return jnp.dot(a.astype(_BF16), b.astype(_BF16), preferred_element_type=_F32)


def _mm_body(x_ref, w_ref, o_ref):
    o_ref[...] = _bdot(x_ref[...], w_ref[...])


def pmatmul(x, w, bm=256, bn=512):
    M, K = x.shape
    _, N = w.shape
    if M % bm:
        bm = M
    if N % bn:
        bn = N
    return pl.pallas_call(
        _mm_body,
        grid=(M // bm, N // bn),
        in_specs=[
            pl.BlockSpec((bm, K), lambda i, j: (i, 0)),
            pl.BlockSpec((K, bn), lambda i, j: (0, j)),
        ],
        out_specs=pl.BlockSpec((bm, bn), lambda i, j: (i, j)),
        out_shape=jax.ShapeDtypeStruct((M, N), _F32),
    )(x, w)


def _rms_body(x_ref, g_ref, o_ref):
    x = x_ref[...]
    o_ref[...] = x * jax.lax.rsqrt(jnp.mean(x * x, axis=-1, keepdims=True) + EPS) * g_ref[...]


def prmsnorm(x, g, bm=256):
    M, K = x.shape
    if M % bm:
        bm = M
    return pl.pallas_call(
        _rms_body,
        grid=(M // bm,),
        in_specs=[
            pl.BlockSpec((bm, K), lambda i: (i, 0)),
            pl.BlockSpec((1, K), lambda i: (0, 0)),
        ],
        out_specs=pl.BlockSpec((bm, K), lambda i: (i, 0)),
        out_shape=jax.ShapeDtypeStruct((M, K), _F32),
    )(x, g[None, :])


def _mlp_body(x_ref, w1_ref, w3_ref, w2_ref, o_ref):
    i = pl.program_id(1)
    x = x_ref[...]
    h = jax.nn.silu(_bdot(x, w1_ref[...]))
    h = h * _bdot(x, w3_ref[...])
    acc = _bdot(h, w2_ref[...])

    @pl.when(i == 0)
    def _():
        o_ref[...] = acc

    @pl.when(i > 0)
    def _():
        o_ref[...] += acc


def pmlp(x, w1, w3, w2, bm=256, bi=512):
    M, K = x.shape
    I = w1.shape[1]
    return pl.pallas_call(
        _mlp_body,
        grid=(M // bm, I // bi),
        in_specs=[
            pl.BlockSpec((bm, K), lambda i, j: (i, 0)),
            pl.BlockSpec((K, bi), lambda i, j: (0, j)),
            pl.BlockSpec((K, bi), lambda i, j: (0, j)),
            pl.BlockSpec((bi, K), lambda i, j: (j, 0)),
        ],
        out_specs=pl.BlockSpec((bm, K), lambda i, j: (i, 0)),
        out_shape=jax.ShapeDtypeStruct((M, K), _F32),
    )(x, w1, w3, w2)


_QK_DIM = QK_NOPE + QK_ROPE
_SCALE = _QK_DIM ** -0.5


def _attn_body(q_ref, k_ref, v_ref, o_ref, *, bq, bk, dv):
    qb = pl.program_id(1)
    q = q_ref[0].astype(_BF16)

    def body(j, carry):
        m, l, acc = carry
        k = k_ref[0, pl.ds(j * bk, bk), :].astype(_BF16)
        v = v_ref[0, pl.ds(j * bk, bk), :].astype(_BF16)
        s = jax.lax.dot_general(q, k, (((1,), (1,)), ((), ())),
                                preferred_element_type=_F32) * _SCALE
        rows = qb * bq + jax.lax.broadcasted_iota(jnp.int32, s.shape, 0)
        cols = j * bk + jax.lax.broadcasted_iota(jnp.int32, s.shape, 1)
        s = jnp.where(cols <= rows, s, -1e30)
        mj = jnp.max(s, axis=-1, keepdims=True)
        mn = jnp.maximum(m, mj)
        p = jnp.exp(s - mn)
        alpha = jnp.exp(m - mn)
        l = l * alpha + jnp.sum(p, axis=-1, keepdims=True)
        acc = acc * alpha + jnp.dot(p.astype(_BF16), v, preferred_element_type=_F32)
        return mn, l, acc

    nk = (qb + 1) * bq // bk
    m0 = jnp.full((bq, 1), -1e30, _F32)
    l0 = jnp.zeros((bq, 1), _F32)
    a0 = jnp.zeros((bq, dv), _F32)
    m, l, acc = jax.lax.fori_loop(0, nk, body, (m0, l0, a0))
    o_ref[0] = acc / l


def pattention(qf, k, v, bq=256, bk=256):
    H, s_len, dq = qf.shape
    dv = v.shape[-1]
    return pl.pallas_call(
        functools.partial(_attn_body, bq=bq, bk=bk, dv=dv),
        grid=(H, s_len // bq),
        in_specs=[
            pl.BlockSpec((1, bq, dq), lambda h, i: (h, i, 0)),
            pl.BlockSpec((1, s_len, dq), lambda h, i: (h, 0, 0)),
            pl.BlockSpec((1, s_len, dv), lambda h, i: (h, 0, 0)),
        ],
        out_specs=pl.BlockSpec((1, bq, dv), lambda h, i: (h, i, 0)),
        out_shape=jax.ShapeDtypeStruct((H, s_len, dv), _F32),
    )(qf, k, v)


def _expert_body(b_ref, w1_ref, w3_ref, w2_ref, o_ref):
    x = b_ref[0]
    h = jax.nn.silu(_bdot(x, w1_ref[0]))
    h = h * _bdot(x, w3_ref[0])
    o_ref[0] = _bdot(h, w2_ref[0])


def pexperts(buf, w1, w3, w2):
    return pl.pallas_call(
        _expert_body,
        grid=(N_EXP,),
        in_specs=[
            pl.BlockSpec((1, CAP, DIM), lambda e: (e, 0, 0)),
            pl.BlockSpec((1, DIM, MOE_INTER), lambda e: (e, 0, 0)),
            pl.BlockSpec((1, DIM, MOE_INTER), lambda e: (e, 0, 0)),
            pl.BlockSpec((1, MOE_INTER, DIM), lambda e: (e, 0, 0)),
        ],
        out_specs=pl.BlockSpec((1, CAP, DIM), lambda e: (e, 0, 0)),
        out_shape=jax.ShapeDtypeStruct((N_EXP, CAP, DIM), _F32),
    )(buf, w1, w3, w2)


def _head_body(x_ref, g_ref, w_ref, o_ref):
    x = x_ref[...]
    xn = x * jax.lax.rsqrt(jnp.mean(x * x, axis=-1, keepdims=True) + EPS) * g_ref[...]
    o_ref[...] = jnp.dot(xn, w_ref[...], preferred_element_type=_F32)


def phead(x_last, g, w, bn=1280):
    M = 8
    xp = jnp.zeros((M, DIM), _F32).at[0].set(x_last)
    out = pl.pallas_call(
        _head_body,
        grid=(VOCAB // bn,),
        in_specs=[
            pl.BlockSpec((M, DIM), lambda j: (0, 0)),
            pl.BlockSpec((1, DIM), lambda j: (0, 0)),
            pl.BlockSpec((DIM, bn), lambda j: (0, j)),
        ],
        out_specs=pl.BlockSpec((M, bn), lambda j: (0, j)),
        out_shape=jax.ShapeDtypeStruct((M, VOCAB), _F32),
    )(xp, g[None, :], w)
    return out[:1]


def _rope(x, cos, sin):
    s_len, h, r = x.shape
    x2 = x.reshape(s_len, h, r // 2, 2)
    x0, x1 = x2[..., 0], x2[..., 1]
    c = cos[:, None, :]
    sn = sin[:, None, :]
    return jnp.stack([x0 * c - x1 * sn, x0 * sn + x1 * c], -1).reshape(s_len, h, r)


def _mla(x, lp, cos, sin):
    s_len = x.shape[0]
    q = pmatmul(x, lp['wq']).reshape(s_len, N_HEADS, _QK_DIM)
    kv = pmatmul(x, lp['wkv_a'])
    kv_c, k_pe = kv[:, :KV_RANK], kv[:, KV_RANK:]
    kv_cn = prmsnorm(kv_c, lp['kv_norm'])
    kvb = pmatmul(kv_cn, lp['wkv_b']).reshape(s_len, N_HEADS, QK_NOPE + V_HEAD)
    q_nope, q_pe = q[..., :QK_NOPE], q[..., QK_NOPE:]
    q_pe = _rope(q_pe, cos, sin)
    k_pe = _rope(k_pe[:, None, :], cos, sin)
    k_nope, v = kvb[..., :QK_NOPE], kvb[..., QK_NOPE:]
    qf = jnp.concatenate([q_nope, q_pe], -1).transpose(1, 0, 2)
    k = jnp.concatenate(
        [k_nope, jnp.broadcast_to(k_pe, (s_len, N_HEADS, QK_ROPE))], -1
    ).transpose(1, 0, 2)
    vt = v.transpose(1, 0, 2)
    o = pattention(qf, k, vt)
    o = o.transpose(1, 0, 2).reshape(s_len, N_HEADS * V_HEAD)
    return pmatmul(o, lp['wo'])


def _moe(x, lp):
    T = x.shape[0]
    logits = pmatmul(x, lp['gate_w'].T)
    scores = jax.nn.softmax(logits, axis=-1)
    topw, topi = jax.lax.top_k(scores, TOPK)
    flat_e = topi.reshape(-1)
    flat_w = topw.reshape(-1)
    oh = jax.nn.one_hot(flat_e, N_EXP, dtype=jnp.int32)
    pos = jnp.cumsum(oh, axis=0)[jnp.arange(T * TOPK), flat_e] - 1
    valid = (pos < CAP).astype(_F32)
    safe_pos = jnp.minimum(pos, CAP - 1)
    vals = jnp.repeat(x, TOPK, axis=0) * valid[:, None]
    buf = jnp.zeros((N_EXP, CAP, DIM), _F32).at[flat_e, safe_pos].add(vals)
    eo = pexperts(buf, lp['e_w1'], lp['e_w3'], lp['e_w2'])
    gathered = eo[flat_e, safe_pos] * (flat_w * valid)[:, None]
    y = gathered.reshape(T, TOPK, DIM).sum(axis=1)
    z = pmlp(x, lp['s_w1'], lp['s_w3'], lp['s_w2'])
    return y + z


def kernel(params, input_ids):
    b, s_len = input_ids.shape
    ids = input_ids.reshape(-1)
    h = params['embed'][ids]
    inv = 1.0 / (ROPE_THETA ** (jnp.arange(0, QK_ROPE, 2, dtype=_F32) / QK_ROPE))
    t = jnp.arange(s_len, dtype=_F32)
    freqs = jnp.outer(t, inv)
    cos, sin = jnp.cos(freqs), jnp.sin(freqs)
    for li, lp in enumerate(params['layers']):
        x = prmsnorm(h, lp['attn_norm'])
        h = h + _mla(x, lp, cos, sin)
        x = prmsnorm(h, lp['ffn_norm'])
        if li < N_DENSE:
            f = pmlp(x, lp['w1'], lp['w3'], lp['w2'])
        else:
            f = _moe(x, lp)
        h = h + f
    logits = phead(h[-1], params['final_norm'], params['head'])
    return logits


# speedup vs baseline: 1.3322x; 1.3322x over previous
import functools

import jax
import jax.numpy as jnp
from jax.experimental import pallas as pl
from jax.experimental.pallas import tpu as pltpu

VOCAB = 32000
DIM = 2048
N_LAYERS = 2
N_DENSE = 1
N_HEADS = 16
QK_NOPE = 128
QK_ROPE = 32
V_HEAD = 128
KV_RANK = 512
INTER = 4096
MOE_INTER = 512
N_EXP = 32
TOPK = 4
N_SHARED = 2
ROPE_THETA = 10000.0
EPS = 1e-6
CAP = 512
S = 2048

_F32 = jnp.float32


_BF16 = jnp.bfloat16


def _bdot(a, b):
    return jnp.dot(a.astype(_BF16), b.astype(_BF16), preferred_element_type=_F32)


def _mm_body(x_ref, w_ref, o_ref):
    o_ref[...] = _bdot(x_ref[...], w_ref[...])


def pmatmul(x, w, bm=512, bn=None):
    M, K = x.shape
    _, N = w.shape
    if M % bm:
        bm = M
    if bn is None:
        bn = N if K * N * 4 <= 24 * 1024 * 1024 else 512
    if N % bn:
        bn = N
    return pl.pallas_call(
        _mm_body,
        grid=(N // bn, M // bm),
        in_specs=[
            pl.BlockSpec((bm, K), lambda j, i: (i, 0)),
            pl.BlockSpec((K, bn), lambda j, i: (0, j)),
        ],
        out_specs=pl.BlockSpec((bm, bn), lambda j, i: (i, j)),
        out_shape=jax.ShapeDtypeStruct((M, N), _F32),
    )(x, w)


def _rms_body(x_ref, g_ref, o_ref):
    x = x_ref[...]
    o_ref[...] = x * jax.lax.rsqrt(jnp.mean(x * x, axis=-1, keepdims=True) + EPS) * g_ref[...]


def prmsnorm(x, g, bm=256):
    M, K = x.shape
    if M % bm:
        bm = M
    return pl.pallas_call(
        _rms_body,
        grid=(M // bm,),
        in_specs=[
            pl.BlockSpec((bm, K), lambda i: (i, 0)),
            pl.BlockSpec((1, K), lambda i: (0, 0)),
        ],
        out_specs=pl.BlockSpec((bm, K), lambda i: (i, 0)),
        out_shape=jax.ShapeDtypeStruct((M, K), _F32),
    )(x, g[None, :])


def _mlp_body(x_ref, w1_ref, w3_ref, w2_ref, o_ref):
    i = pl.program_id(1)
    x = x_ref[...]
    h = jax.nn.silu(_bdot(x, w1_ref[...]))
    h = h * _bdot(x, w3_ref[...])
    acc = _bdot(h, w2_ref[...])

    @pl.when(i == 0)
    def _():
        o_ref[...] = acc

    @pl.when(i > 0)
    def _():
        o_ref[...] += acc


def pmlp(x, w1, w3, w2, bm=2048, bi=128):
    M, K = x.shape
    I = w1.shape[1]
    return pl.pallas_call(
        _mlp_body,
        grid=(M // bm, I // bi),
        in_specs=[
            pl.BlockSpec((bm, K), lambda i, j: (i, 0)),
            pl.BlockSpec((K, bi), lambda i, j: (0, j)),
            pl.BlockSpec((K, bi), lambda i, j: (0, j)),
            pl.BlockSpec((bi, K), lambda i, j: (j, 0)),
        ],
        out_specs=pl.BlockSpec((bm, K), lambda i, j: (i, 0)),
        out_shape=jax.ShapeDtypeStruct((M, K), _F32),
    )(x, w1, w3, w2)


_QK_DIM = QK_NOPE + QK_ROPE
_SCALE = _QK_DIM ** -0.5


def _attn_body(q_ref, k_ref, v_ref, o_ref, *, bq, bk, dv):
    qb = pl.program_id(1)
    q = q_ref[0].astype(_BF16)
    k = k_ref[0].astype(_BF16)
    s = jax.lax.dot_general(q, k, (((1,), (1,)), ((), ())),
                            preferred_element_type=_F32) * _SCALE
    rows = qb * bq + jax.lax.broadcasted_iota(jnp.int32, s.shape, 0)
    cols = jax.lax.broadcasted_iota(jnp.int32, s.shape, 1)
    s = jnp.where(cols <= rows, s, -1e30)
    m = jnp.max(s, axis=-1, keepdims=True)
    e = jnp.exp(s - m)
    p = (e / jnp.sum(e, axis=-1, keepdims=True)).astype(_BF16)
    o_ref[0] = jnp.dot(p, v_ref[0].astype(_BF16), preferred_element_type=_F32)


def pattention(qf, k, v, bq=256, bk=256):
    H, s_len, dq = qf.shape
    dv = v.shape[-1]
    return pl.pallas_call(
        functools.partial(_attn_body, bq=bq, bk=bk, dv=dv),
        grid=(H, s_len // bq),
        in_specs=[
            pl.BlockSpec((1, bq, dq), lambda h, i: (h, i, 0)),
            pl.BlockSpec((1, s_len, dq), lambda h, i: (h, 0, 0)),
            pl.BlockSpec((1, s_len, dv), lambda h, i: (h, 0, 0)),
        ],
        out_specs=pl.BlockSpec((1, bq, dv), lambda h, i: (h, i, 0)),
        out_shape=jax.ShapeDtypeStruct((H, s_len, dv), _F32),
    )(qf, k, v)


def _expert_body(b_ref, w1_ref, w3_ref, w2_ref, o_ref):
    x = b_ref[0]
    h = jax.nn.silu(_bdot(x, w1_ref[0]))
    h = h * _bdot(x, w3_ref[0])
    o_ref[0] = _bdot(h, w2_ref[0])


def pexperts(buf, w1, w3, w2):
    return pl.pallas_call(
        _expert_body,
        grid=(N_EXP,),
        in_specs=[
            pl.BlockSpec((1, CAP, DIM), lambda e: (e, 0, 0)),
            pl.BlockSpec((1, DIM, MOE_INTER), lambda e: (e, 0, 0)),
            pl.BlockSpec((1, DIM, MOE_INTER), lambda e: (e, 0, 0)),
            pl.BlockSpec((1, MOE_INTER, DIM), lambda e: (e, 0, 0)),
        ],
        out_specs=pl.BlockSpec((1, CAP, DIM), lambda e: (e, 0, 0)),
        out_shape=jax.ShapeDtypeStruct((N_EXP, CAP, DIM), _F32),
    )(buf, w1, w3, w2)


def _head_body(x_ref, g_ref, w_ref, o_ref):
    x = x_ref[...]
    xn = x * jax.lax.rsqrt(jnp.mean(x * x, axis=-1, keepdims=True) + EPS) * g_ref[...]
    o_ref[...] = jnp.dot(xn, w_ref[...], preferred_element_type=_F32)


def phead(x_last, g, w, bn=1280):
    M = 8
    xp = jnp.zeros((M, DIM), _F32).at[0].set(x_last)
    out = pl.pallas_call(
        _head_body,
        grid=(VOCAB // bn,),
        in_specs=[
            pl.BlockSpec((M, DIM), lambda j: (0, 0)),
            pl.BlockSpec((1, DIM), lambda j: (0, 0)),
            pl.BlockSpec((DIM, bn), lambda j: (0, j)),
        ],
        out_specs=pl.BlockSpec((M, bn), lambda j: (0, j)),
        out_shape=jax.ShapeDtypeStruct((M, VOCAB), _F32),
    )(xp, g[None, :], w)
    return out[:1]


def _rope(x, cos, sin):
    s_len, h, r = x.shape
    x2 = x.reshape(s_len, h, r // 2, 2)
    x0, x1 = x2[..., 0], x2[..., 1]
    c = cos[:, None, :]
    sn = sin[:, None, :]
    return jnp.stack([x0 * c - x1 * sn, x0 * sn + x1 * c], -1).reshape(s_len, h, r)


def _mla(x, lp, cos, sin):
    s_len = x.shape[0]
    q = pmatmul(x, lp['wq']).reshape(s_len, N_HEADS, _QK_DIM)
    kv = pmatmul(x, lp['wkv_a'])
    kv_c, k_pe = kv[:, :KV_RANK], kv[:, KV_RANK:]
    kv_cn = prmsnorm(kv_c, lp['kv_norm'])
    kvb = pmatmul(kv_cn, lp['wkv_b']).reshape(s_len, N_HEADS, QK_NOPE + V_HEAD)
    q_nope, q_pe = q[..., :QK_NOPE], q[..., QK_NOPE:]
    q_pe = _rope(q_pe, cos, sin)
    k_pe = _rope(k_pe[:, None, :], cos, sin)
    k_nope, v = kvb[..., :QK_NOPE], kvb[..., QK_NOPE:]
    qf = jnp.concatenate([q_nope, q_pe], -1).transpose(1, 0, 2)
    k = jnp.concatenate(
        [k_nope, jnp.broadcast_to(k_pe, (s_len, N_HEADS, QK_ROPE))], -1
    ).transpose(1, 0, 2)
    vt = v.transpose(1, 0, 2)
    o = pattention(qf, k, vt)
    o = o.transpose(1, 0, 2).reshape(s_len, N_HEADS * V_HEAD)
    return pmatmul(o, lp['wo'])


def _moe(x, lp):
    T = x.shape[0]
    logits = pmatmul(x, lp['gate_w'].T)
    scores = jax.nn.softmax(logits, axis=-1)
    topw, topi = jax.lax.top_k(scores, TOPK)
    hist = jax.nn.one_hot(topi, N_EXP, dtype=jnp.int32).sum(axis=1)
    cum_excl = jnp.cumsum(hist, axis=0) - hist
    pos = jnp.take_along_axis(cum_excl, topi, axis=1).reshape(-1)
    flat_e = topi.reshape(-1)
    flat_w = topw.reshape(-1)
    valid = pos < CAP
    safe_pos = jnp.minimum(pos, CAP - 1)
    slot = jnp.where(valid, flat_e * CAP + safe_pos, N_EXP * CAP)
    flat_t = jnp.arange(T * TOPK, dtype=jnp.int32) // TOPK
    src = jnp.full((N_EXP * CAP + 1,), T, jnp.int32).at[slot].set(flat_t)
    x_pad = jnp.concatenate([x, jnp.zeros((1, DIM), _F32)], axis=0)
    buf = x_pad[src[:N_EXP * CAP]].reshape(N_EXP, CAP, DIM)
    eo = pexperts(buf, lp['e_w1'], lp['e_w3'], lp['e_w2'])
    gathered = eo.reshape(N_EXP * CAP, DIM)[flat_e * CAP + safe_pos]
    gathered = gathered * (flat_w * valid.astype(_F32))[:, None]
    y = gathered.reshape(T, TOPK, DIM).sum(axis=1)
    z = pmlp(x, lp['s_w1'], lp['s_w3'], lp['s_w2'])
    return y + z


def kernel(params, input_ids):
    b, s_len = input_ids.shape
    ids = input_ids.reshape(-1)
    h = params['embed'][ids]
    inv = 1.0 / (ROPE_THETA ** (jnp.arange(0, QK_ROPE, 2, dtype=_F32) / QK_ROPE))
    t = jnp.arange(s_len, dtype=_F32)
    freqs = jnp.outer(t, inv)
    cos, sin = jnp.cos(freqs), jnp.sin(freqs)
    for li, lp in enumerate(params['layers']):
        x = prmsnorm(h, lp['attn_norm'])
        h = h + _mla(x, lp, cos, sin)
        x = prmsnorm(h, lp['ffn_norm'])
        if li < N_DENSE:
            f = pmlp(x, lp['w1'], lp['w3'], lp['w2'])
        else:
            f = _moe(x, lp)
        h = h + f
    logits = phead(h[-1], params['final_norm'], params['head'])
    return logits
